# baseline (speedup 1.0000x reference)
import jax
import jax.numpy as jnp
from jax import lax
from jax.experimental import pallas as pl
from jax.experimental.pallas import tpu as pltpu
from jax.experimental.pallas import tpu_sc as plsc

_VP = 6.0
_VS = 6.0 / 1.73
_NC, _NS, _L = 2, 16, 16
_NW = _NC * _NS
_C = 2000
_G = 80
_W = 8
_AB = 4000


def _rsqrt(x):
    i = plsc.bitcast(x, jnp.int32)
    y = plsc.bitcast(jnp.int32(0x5F3759DF) - (i >> 1), jnp.float32)
    for _ in range(3):
        y = y * (1.5 - 0.5 * x * y * y)
    return y


def _make_body(nev, nch, kmax):
    nblk = nev // _AB

    def _body(eloc, etime, sidx, eidx, pt, stloc, stdt, out, pk,
              eidx_v0, eidx_v1, idx_v0, idx_v1, sidx_v0, sidx_v1,
              pt_v0, pt_v1, rows_v0, rows_v1, out_v, stloc_v, stdt_v,
              pa_loc, pa_t, pa_out, gsem0, gsem1):
        eidx_v = (eidx_v0, eidx_v1)
        idx_v = (idx_v0, idx_v1)
        sidx_v = (sidx_v0, sidx_v1)
        pt_v = (pt_v0, pt_v1)
        rows_v = (rows_v0, rows_v1)
        gsem = (gsem0, gsem1)
        sid = lax.axis_index("s")
        wid = sid * _NC + lax.axis_index("c")
        pltpu.sync_copy(stloc, stloc_v)
        pltpu.sync_copy(stdt, stdt_v)
        zeros = jnp.zeros((_L,), jnp.int32)
        iota = lax.iota(jnp.int32, _L)

        for k in range((nblk + _NS - 1) // _NS):
            bid = k * _NS + sid

            @pl.when(bid < nblk)
            def _():
                ev0 = bid * _AB
                pltpu.sync_copy(eloc.at[pl.ds(ev0, _AB)], pa_loc)
                pltpu.sync_copy(etime.at[pl.ds(ev0, _AB)], pa_t)

                def ablock(v, c2):
                    w = v * _L + iota
                    ev = w >> 2
                    cp = w & 3
                    a = plsc.load_gather(pa_loc, [ev, jnp.minimum(cp, 2)])
                    b = plsc.load_gather(pa_t, [ev])
                    val = jnp.where(cp == 3, b, a)
                    plsc.store_scatter(pa_out, [w >> 3, w & 7], val)
                    return c2

                lax.fori_loop(0, _AB * 4 // _L, ablock, 0)
                pltpu.sync_copy(pa_out, pk.at[pl.ds(bid * (_AB // 2), _AB // 2)])

        plsc.subcore_barrier()

        def fire(k, b):
            cid = k * _NW + wid

            @pl.when(cid < nch)
            def _():
                base = cid * _C
                pltpu.sync_copy(eidx.at[pl.ds(base, _C)], eidx_v[b])

                def half_body(j, c2):
                    s = j * _L
                    idx_v[b][pl.ds(s, _L)] = eidx_v[b][pl.ds(s, _L)] >> 1
                    return c2

                lax.fori_loop(0, _C // _L, half_body, 0)
                for j in range(_C // _G):
                    pltpu.async_copy(
                        pk.at[idx_v[b].at[pl.ds(j * _G, _G)]],
                        rows_v[b].at[pl.ds(j * _G, _G)],
                        gsem[b],
                    )
                pltpu.sync_copy(sidx.at[pl.ds(base, _C)], sidx_v[b])
                pltpu.sync_copy(pt.at[pl.ds(base, _C)], pt_v[b])

        def consume(k, b):
            cid = k * _NW + wid

            @pl.when(cid < nch)
            def _():
                base = cid * _C
                pltpu.make_async_copy(
                    pk.at[pl.ds(0, _C)], rows_v[b], gsem[b]).wait()

                def lane_body(j, c2):
                    s = j * _L
                    row = s + iota
                    sv = sidx_v[b][pl.ds(s, _L)]
                    ptv = pt_v[b][pl.ds(s, _L)]
                    half = (eidx_v[b][pl.ds(s, _L)] & 1) * 4
                    ex = plsc.load_gather(rows_v[b], [row, half])
                    ey = plsc.load_gather(rows_v[b], [row, half + 1])
                    ez = plsc.load_gather(rows_v[b], [row, half + 2])
                    et = plsc.load_gather(rows_v[b], [row, half + 3])
                    sx = plsc.load_gather(stloc_v, [sv, zeros])
                    sy = plsc.load_gather(stloc_v, [sv, zeros + 1])
                    sz = plsc.load_gather(stloc_v, [sv, zeros + 2])
                    sdt = plsc.load_gather(stdt_v, [sv, ptv])
                    dx = ex - sx
                    dy = ey - sy
                    dz = ez - sz
                    dsq = dx * dx + dy * dy + dz * dz
                    dist = dsq * _rsqrt(jnp.maximum(dsq, 1e-12))
                    iv = jnp.where(ptv == 0, 1.0 / _VP, 1.0 / _VS)
                    out_v[pl.ds(s, _L)] = et + dist * iv + sdt
                    return c2

                lax.fori_loop(0, _C // _L, lane_body, 0)
                pltpu.sync_copy(out_v, out.at[pl.ds(base, _C)])

        fire(0, 0)

        def pair_body(kk, carry):
            k0 = kk * 2
            fire(k0 + 1, 1)
            consume(k0, 0)
            fire(k0 + 2, 0)
            consume(k0 + 1, 1)
            return carry

        lax.fori_loop(0, kmax // 2, pair_body, 0)

    return _body


def _sc_travel_time(eloc, etime, sidx, eidx, pt, stloc, stdt):
    n = sidx.shape[0]
    nev = eloc.shape[0]
    nch = n // _C
    kmax = (nch + _NW - 1) // _NW
    kmax += kmax % 2
    nst = stloc.shape[0]
    mesh = plsc.VectorSubcoreMesh(core_axis_name="c", subcore_axis_name="s")
    out, _ = pl.kernel(
        _make_body(nev, nch, kmax),
        out_type=(
            jax.ShapeDtypeStruct((n,), jnp.float32),
            jax.ShapeDtypeStruct((nev // 2, 2 * 4), jnp.float32),
        ),
        mesh=mesh,
        scratch_types=[
            pltpu.VMEM((_C,), jnp.int32),
            pltpu.VMEM((_C,), jnp.int32),
            pltpu.VMEM((_C,), jnp.int32),
            pltpu.VMEM((_C,), jnp.int32),
            pltpu.VMEM((_C,), jnp.int32),
            pltpu.VMEM((_C,), jnp.int32),
            pltpu.VMEM((_C,), jnp.int32),
            pltpu.VMEM((_C,), jnp.int32),
            pltpu.VMEM((_C, _W), jnp.float32),
            pltpu.VMEM((_C, _W), jnp.float32),
            pltpu.VMEM((_C,), jnp.float32),
            pltpu.VMEM((nst, 3), jnp.float32),
            pltpu.VMEM((nst, 2), jnp.float32),
            pltpu.VMEM((_AB, 3), jnp.float32),
            pltpu.VMEM((_AB,), jnp.float32),
            pltpu.VMEM((_AB // 2, _W), jnp.float32),
            pltpu.SemaphoreType.DMA,
            pltpu.SemaphoreType.DMA,
        ],
        compiler_params=pltpu.CompilerParams(
            needs_layout_passes=False, use_tc_tiling_on_sc=False),
    )(eloc, etime, sidx, eidx, pt, stloc, stdt)
    return out


def kernel(station_index, event_index, phase_type, phase_weight,
           station_loc_w, station_dt_w, event_loc_w, event_time_w):
    out = _sc_travel_time(event_loc_w, event_time_w.reshape(-1),
                          station_index, event_index, phase_type,
                          station_loc_w, station_dt_w)
    return out[:, None]


# speedup vs baseline: 1.2539x; 1.2539x over previous
import jax
import jax.numpy as jnp
from jax import lax
from jax.experimental import pallas as pl
from jax.experimental.pallas import tpu as pltpu
from jax.experimental.pallas import tpu_sc as plsc

_VP = 6.0
_VS = 6.0 / 1.73
_NC, _NS, _L = 2, 16, 16
_NW = _NC * _NS
_C = 4000
_G = 80
_W = 8


def _rsqrt(x):
    i = plsc.bitcast(x, jnp.int32)
    y = plsc.bitcast(jnp.int32(0x5F3759DF) - (i >> 1), jnp.float32)
    for _ in range(3):
        y = y * (1.5 - 0.5 * x * y * y)
    return y


def _make_body(nch, kmax):
    def _body(packed, sidx, eidx, pt, stloc, stdt, out,
              idx_v0, idx_v1, sidx_v0, sidx_v1, pt_v0, pt_v1,
              rows_v0, rows_v1, out_v, stloc_v, stdt_v, gsem0, gsem1):
        idx_v = (idx_v0, idx_v1)
        sidx_v = (sidx_v0, sidx_v1)
        pt_v = (pt_v0, pt_v1)
        rows_v = (rows_v0, rows_v1)
        gsem = (gsem0, gsem1)
        wid = lax.axis_index("s") * _NC + lax.axis_index("c")
        pltpu.sync_copy(stloc, stloc_v)
        pltpu.sync_copy(stdt, stdt_v)
        zeros = jnp.zeros((_L,), jnp.int32)

        def fire(k, b):
            cid = k * _NW + wid

            @pl.when(cid < nch)
            def _():
                base = cid * _C
                pltpu.sync_copy(eidx.at[pl.ds(base, _C)], idx_v[b])
                for j in range(_C // _G):
                    pltpu.async_copy(
                        packed.at[idx_v[b].at[pl.ds(j * _G, _G)]],
                        rows_v[b].at[pl.ds(j * _G, _G)],
                        gsem[b],
                    )
                pltpu.sync_copy(sidx.at[pl.ds(base, _C)], sidx_v[b])
                pltpu.sync_copy(pt.at[pl.ds(base, _C)], pt_v[b])

        def consume(k, b):
            cid = k * _NW + wid

            @pl.when(cid < nch)
            def _():
                base = cid * _C
                pltpu.make_async_copy(
                    packed.at[pl.ds(0, _C)], rows_v[b], gsem[b]).wait()

                def lane_body(j, c2):
                    s = j * _L
                    row = s + lax.iota(jnp.int32, _L)
                    sv = sidx_v[b][pl.ds(s, _L)]
                    ptv = pt_v[b][pl.ds(s, _L)]
                    ex = plsc.load_gather(rows_v[b], [row, zeros])
                    ey = plsc.load_gather(rows_v[b], [row, zeros + 1])
                    ez = plsc.load_gather(rows_v[b], [row, zeros + 2])
                    et = plsc.load_gather(rows_v[b], [row, zeros + 3])
                    sx = plsc.load_gather(stloc_v, [sv, zeros])
                    sy = plsc.load_gather(stloc_v, [sv, zeros + 1])
                    sz = plsc.load_gather(stloc_v, [sv, zeros + 2])
                    sdt = plsc.load_gather(stdt_v, [sv, ptv])
                    dx = ex - sx
                    dy = ey - sy
                    dz = ez - sz
                    dsq = dx * dx + dy * dy + dz * dz
                    dist = dsq * _rsqrt(jnp.maximum(dsq, 1e-12))
                    iv = jnp.where(ptv == 0, 1.0 / _VP, 1.0 / _VS)
                    out_v[pl.ds(s, _L)] = et + dist * iv + sdt
                    return c2

                lax.fori_loop(0, _C // _L, lane_body, 0)
                pltpu.sync_copy(out_v, out.at[pl.ds(base, _C)])

        fire(0, 0)

        def pair_body(kk, carry):
            k0 = kk * 2
            fire(k0 + 1, 1)
            consume(k0, 0)
            fire(k0 + 2, 0)
            consume(k0 + 1, 1)
            return carry

        lax.fori_loop(0, kmax // 2, pair_body, 0)

    return _body


def _sc_travel_time(packed, sidx, eidx, pt, stloc, stdt):
    n = sidx.shape[0]
    nch = n // _C
    kmax = (nch + _NW - 1) // _NW
    kmax += kmax % 2
    nst = stloc.shape[0]
    mesh = plsc.VectorSubcoreMesh(core_axis_name="c", subcore_axis_name="s")
    return pl.kernel(
        _make_body(nch, kmax),
        out_type=jax.ShapeDtypeStruct((n,), jnp.float32),
        mesh=mesh,
        scratch_types=[
            pltpu.VMEM((_C,), jnp.int32),
            pltpu.VMEM((_C,), jnp.int32),
            pltpu.VMEM((_C,), jnp.int32),
            pltpu.VMEM((_C,), jnp.int32),
            pltpu.VMEM((_C,), jnp.int32),
            pltpu.VMEM((_C,), jnp.int32),
            pltpu.VMEM((_C, _W), jnp.float32),
            pltpu.VMEM((_C, _W), jnp.float32),
            pltpu.VMEM((_C,), jnp.float32),
            pltpu.VMEM((nst, 3), jnp.float32),
            pltpu.VMEM((nst, 2), jnp.float32),
            pltpu.SemaphoreType.DMA,
            pltpu.SemaphoreType.DMA,
        ],
        compiler_params=pltpu.CompilerParams(
            needs_layout_passes=False, use_tc_tiling_on_sc=False),
    )(packed, sidx, eidx, pt, stloc, stdt)


def kernel(station_index, event_index, phase_type, phase_weight,
           station_loc_w, station_dt_w, event_loc_w, event_time_w):
    packed = jnp.concatenate(
        [event_loc_w, event_time_w,
         jnp.zeros((event_loc_w.shape[0], _W - 4), jnp.float32)], axis=1)
    out = _sc_travel_time(packed, station_index, event_index, phase_type,
                          station_loc_w, station_dt_w)
    return out[:, None]


# speedup vs baseline: 1.6504x; 1.3162x over previous
import jax
import jax.numpy as jnp
from jax import lax
from jax.experimental import pallas as pl
from jax.experimental.pallas import tpu as pltpu
from jax.experimental.pallas import tpu_sc as plsc

_VP = 6.0
_VS = 6.0 / 1.73
_NC, _NS, _L = 2, 16, 16
_NW = _NC * _NS
_C = 4000
_G = 80
_W = 8


def _rsqrt(x):
    i = plsc.bitcast(x, jnp.int32)
    y = plsc.bitcast(jnp.int32(0x5F3759DF) - (i >> 1), jnp.float32)
    for _ in range(3):
        y = y * (1.5 - 0.5 * x * y * y)
    return y


def _make_body(nch, kmax):
    def _body(packed, sidx, eidx, pt, stloc, stdt, out,
              idx_v0, idx_v1, sidx_v0, sidx_v1, pt_v0, pt_v1,
              rows_v0, rows_v1, out_v, stloc_v, stdt_v, gsem0, gsem1):
        idx_v = (idx_v0, idx_v1)
        sidx_v = (sidx_v0, sidx_v1)
        pt_v = (pt_v0, pt_v1)
        rows_v = (rows_v0, rows_v1)
        gsem = (gsem0, gsem1)
        wid = lax.axis_index("s") * _NC + lax.axis_index("c")
        pltpu.sync_copy(stloc, stloc_v)
        pltpu.sync_copy(stdt, stdt_v)
        zeros = jnp.zeros((_L,), jnp.int32)

        def fire(k, b):
            cid = k * _NW + wid

            @pl.when(cid < nch)
            def _():
                base = cid * _C
                pltpu.sync_copy(eidx.at[pl.ds(base, _C)], idx_v[b])
                for j in range(_C // _G):
                    pltpu.async_copy(
                        packed.at[idx_v[b].at[pl.ds(j * _G, _G)]],
                        rows_v[b].at[pl.ds(j * _G, _G)],
                        gsem[b],
                    )
                pltpu.sync_copy(sidx.at[pl.ds(base, _C)], sidx_v[b])
                pltpu.sync_copy(pt.at[pl.ds(base, _C)], pt_v[b])

        def consume(k, b):
            cid = k * _NW + wid

            @pl.when(cid < nch)
            def _():
                base = cid * _C
                pltpu.make_async_copy(
                    packed.at[pl.ds(0, _C)], rows_v[b], gsem[b]).wait()

                @plsc.parallel_loop(0, _C, step=_L, unroll=4)
                def lane_body(s):
                    row = s + lax.iota(jnp.int32, _L)
                    sv = sidx_v[b][pl.ds(s, _L)]
                    ptv = pt_v[b][pl.ds(s, _L)]
                    ex = plsc.load_gather(rows_v[b], [row, zeros])
                    ey = plsc.load_gather(rows_v[b], [row, zeros + 1])
                    ez = plsc.load_gather(rows_v[b], [row, zeros + 2])
                    et = plsc.load_gather(rows_v[b], [row, zeros + 3])
                    sx = plsc.load_gather(stloc_v, [sv, zeros])
                    sy = plsc.load_gather(stloc_v, [sv, zeros + 1])
                    sz = plsc.load_gather(stloc_v, [sv, zeros + 2])
                    sdt = plsc.load_gather(stdt_v, [sv, ptv])
                    dx = ex - sx
                    dy = ey - sy
                    dz = ez - sz
                    dsq = dx * dx + dy * dy + dz * dz
                    dist = dsq * _rsqrt(jnp.maximum(dsq, 1e-12))
                    iv = jnp.where(ptv == 0, 1.0 / _VP, 1.0 / _VS)
                    out_v[pl.ds(s, _L)] = et + dist * iv + sdt

                pltpu.sync_copy(out_v, out.at[pl.ds(base, _C)])

        fire(0, 0)

        def pair_body(kk, carry):
            k0 = kk * 2
            fire(k0 + 1, 1)
            consume(k0, 0)
            fire(k0 + 2, 0)
            consume(k0 + 1, 1)
            return carry

        lax.fori_loop(0, kmax // 2, pair_body, 0)

    return _body


def _sc_travel_time(packed, sidx, eidx, pt, stloc, stdt):
    n = sidx.shape[0]
    nch = n // _C
    kmax = (nch + _NW - 1) // _NW
    kmax += kmax % 2
    nst = stloc.shape[0]
    mesh = plsc.VectorSubcoreMesh(core_axis_name="c", subcore_axis_name="s")
    return pl.kernel(
        _make_body(nch, kmax),
        out_type=jax.ShapeDtypeStruct((n,), jnp.float32),
        mesh=mesh,
        scratch_types=[
            pltpu.VMEM((_C,), jnp.int32),
            pltpu.VMEM((_C,), jnp.int32),
            pltpu.VMEM((_C,), jnp.int32),
            pltpu.VMEM((_C,), jnp.int32),
            pltpu.VMEM((_C,), jnp.int32),
            pltpu.VMEM((_C,), jnp.int32),
            pltpu.VMEM((_C, _W), jnp.float32),
            pltpu.VMEM((_C, _W), jnp.float32),
            pltpu.VMEM((_C,), jnp.float32),
            pltpu.VMEM((nst, 3), jnp.float32),
            pltpu.VMEM((nst, 2), jnp.float32),
            pltpu.SemaphoreType.DMA,
            pltpu.SemaphoreType.DMA,
        ],
        compiler_params=pltpu.CompilerParams(
            needs_layout_passes=False, use_tc_tiling_on_sc=False,
            disable_bounds_checks=True),
    )(packed, sidx, eidx, pt, stloc, stdt)


def kernel(station_index, event_index, phase_type, phase_weight,
           station_loc_w, station_dt_w, event_loc_w, event_time_w):
    packed = jnp.concatenate(
        [event_loc_w, event_time_w,
         jnp.zeros((event_loc_w.shape[0], _W - 4), jnp.float32)], axis=1)
    out = _sc_travel_time(packed, station_index, event_index, phase_type,
                          station_loc_w, station_dt_w)
    return out[:, None]
